# baseline (speedup 1.0000x reference)
import functools

import jax
import jax.numpy as jnp
from jax import lax
from jax.experimental import pallas as pl
from jax.experimental.pallas import tpu as pltpu

_N = 1000
_E = 1600000
_NC = 50
_CH = _E // _NC


def _body(k_smem, ew_ref, ei_ref, ei_out, lp_out, buf, *sems):
    in_sems = sems[:_NC]
    out_sems = sems[_NC:]

    def in_cp(i):
        return pltpu.make_async_copy(
            ei_ref.at[:, pl.ds(i * _CH, _CH)], buf.at[i], in_sems[i]
        )

    def out_cp(i):
        return pltpu.make_async_copy(
            buf.at[i], ei_out.at[:, pl.ds(i * _CH, _CH)], out_sems[i]
        )

    for i in range(_NC):
        in_cp(i).start()

    r0 = ew_ref[0:1, :]
    r1 = ew_ref[1:2, :]
    x = jnp.where(k_smem[0] == 1, r1, r0)
    m = jnp.max(x)
    sum_x = jnp.sum(x)
    s = jnp.sum(jnp.exp(x - m))
    lp_out[0, 0] = sum_x - jnp.float32(_N) * m - jnp.float32(_N) * jnp.log(s)

    for i in range(_NC):
        in_cp(i).wait()
        out_cp(i).start()
    for i in range(_NC):
        out_cp(i).wait()


@jax.jit
def _run(edge_index, edge_weights, k):
    grid_spec = pltpu.PrefetchScalarGridSpec(
        num_scalar_prefetch=1,
        grid=(1,),
        in_specs=[
            pl.BlockSpec((2, _N), lambda i, k_ref: (0, 0)),
            pl.BlockSpec(memory_space=pl.ANY),
        ],
        out_specs=[
            pl.BlockSpec(memory_space=pl.ANY),
            pl.BlockSpec(memory_space=pltpu.SMEM),
        ],
        scratch_shapes=(
            [pltpu.VMEM((_NC, 2, _CH), jnp.int32)]
            + [pltpu.SemaphoreType.DMA] * (2 * _NC)
        ),
    )
    ei_out, lp = pl.pallas_call(
        _body,
        grid_spec=grid_spec,
        out_shape=[
            jax.ShapeDtypeStruct((2, _E), jnp.int32),
            jax.ShapeDtypeStruct((1, 1), jnp.float32),
        ],
        compiler_params=pltpu.CompilerParams(
            dimension_semantics=("arbitrary",),
            vmem_limit_bytes=100 * 1024 * 1024,
        ),
    )(jnp.reshape(k, (1,)).astype(jnp.int32), edge_weights, edge_index)
    return ei_out, lp[0, 0]


def kernel(edge_index, edge_weights, n, num_sample, k):
    return _run(edge_index, edge_weights, k)


# speedup vs baseline: 1.0574x; 1.0574x over previous
import functools

import jax
import jax.numpy as jnp
from jax import lax
from jax.experimental import pallas as pl
from jax.experimental.pallas import tpu as pltpu

_N = 1000
_E = 1600000
_NC = 10
_CH = _E // _NC


def _body(k_smem, ew_ref, ei_ref, ei_out, lp_out, buf, *sems):
    in_sems = sems[:_NC]
    out_sems = sems[_NC:]

    def in_cp(i):
        return pltpu.make_async_copy(
            ei_ref.at[:, pl.ds(i * _CH, _CH)], buf.at[i], in_sems[i]
        )

    def out_cp(i):
        return pltpu.make_async_copy(
            buf.at[i], ei_out.at[:, pl.ds(i * _CH, _CH)], out_sems[i]
        )

    for i in range(_NC):
        in_cp(i).start()

    r0 = ew_ref[0:1, :]
    r1 = ew_ref[1:2, :]
    x = jnp.where(k_smem[0] == 1, r1, r0)
    m = jnp.max(x)
    sum_x = jnp.sum(x)
    s = jnp.sum(jnp.exp(x - m))
    lp_out[0, 0] = sum_x - jnp.float32(_N) * m - jnp.float32(_N) * jnp.log(s)

    for i in range(_NC):
        in_cp(i).wait()
        out_cp(i).start()
    for i in range(_NC):
        out_cp(i).wait()


@jax.jit
def _run(edge_index, edge_weights, k):
    grid_spec = pltpu.PrefetchScalarGridSpec(
        num_scalar_prefetch=1,
        grid=(1,),
        in_specs=[
            pl.BlockSpec((2, _N), lambda i, k_ref: (0, 0)),
            pl.BlockSpec(memory_space=pl.ANY),
        ],
        out_specs=[
            pl.BlockSpec(memory_space=pl.ANY),
            pl.BlockSpec(memory_space=pltpu.SMEM),
        ],
        scratch_shapes=(
            [pltpu.VMEM((_NC, 2, _CH), jnp.int32)]
            + [pltpu.SemaphoreType.DMA] * (2 * _NC)
        ),
    )
    ei_out, lp = pl.pallas_call(
        _body,
        grid_spec=grid_spec,
        out_shape=[
            jax.ShapeDtypeStruct((2, _E), jnp.int32),
            jax.ShapeDtypeStruct((1, 1), jnp.float32),
        ],
        compiler_params=pltpu.CompilerParams(
            dimension_semantics=("arbitrary",),
            vmem_limit_bytes=100 * 1024 * 1024,
        ),
    )(jnp.reshape(k, (1,)).astype(jnp.int32), edge_weights, edge_index)
    return ei_out, lp[0, 0]


def kernel(edge_index, edge_weights, n, num_sample, k):
    return _run(edge_index, edge_weights, k)


# speedup vs baseline: 1.0802x; 1.0216x over previous
import functools

import jax
import jax.numpy as jnp
from jax import lax
from jax.experimental import pallas as pl
from jax.experimental.pallas import tpu as pltpu

_N = 1000
_E = 1600000
_NC = 5
_CH = _E // _NC


def _body(k_smem, ew_ref, ei_ref, ei_out, lp_out, buf, *sems):
    in_sems = sems[:_NC]
    out_sems = sems[_NC:]

    def in_cp(i):
        return pltpu.make_async_copy(
            ei_ref.at[:, pl.ds(i * _CH, _CH)], buf.at[i], in_sems[i]
        )

    def out_cp(i):
        return pltpu.make_async_copy(
            buf.at[i], ei_out.at[:, pl.ds(i * _CH, _CH)], out_sems[i]
        )

    for i in range(_NC):
        in_cp(i).start()

    r0 = ew_ref[0:1, :]
    r1 = ew_ref[1:2, :]
    x = jnp.where(k_smem[0] == 1, r1, r0)
    m = jnp.max(x)
    sum_x = jnp.sum(x)
    s = jnp.sum(jnp.exp(x - m))
    lp_out[0, 0] = sum_x - jnp.float32(_N) * m - jnp.float32(_N) * jnp.log(s)

    for i in range(_NC):
        in_cp(i).wait()
        out_cp(i).start()
    for i in range(_NC):
        out_cp(i).wait()


@jax.jit
def _run(edge_index, edge_weights, k):
    grid_spec = pltpu.PrefetchScalarGridSpec(
        num_scalar_prefetch=1,
        grid=(1,),
        in_specs=[
            pl.BlockSpec((2, _N), lambda i, k_ref: (0, 0)),
            pl.BlockSpec(memory_space=pl.ANY),
        ],
        out_specs=[
            pl.BlockSpec(memory_space=pl.ANY),
            pl.BlockSpec(memory_space=pltpu.SMEM),
        ],
        scratch_shapes=(
            [pltpu.VMEM((_NC, 2, _CH), jnp.int32)]
            + [pltpu.SemaphoreType.DMA] * (2 * _NC)
        ),
    )
    ei_out, lp = pl.pallas_call(
        _body,
        grid_spec=grid_spec,
        out_shape=[
            jax.ShapeDtypeStruct((2, _E), jnp.int32),
            jax.ShapeDtypeStruct((1, 1), jnp.float32),
        ],
        compiler_params=pltpu.CompilerParams(
            dimension_semantics=("arbitrary",),
            vmem_limit_bytes=100 * 1024 * 1024,
        ),
    )(jnp.reshape(k, (1,)).astype(jnp.int32), edge_weights, edge_index)
    return ei_out, lp[0, 0]


def kernel(edge_index, edge_weights, n, num_sample, k):
    return _run(edge_index, edge_weights, k)
